# baseline (speedup 1.0000x reference)
import functools

import jax
import jax.numpy as jnp
from jax.experimental import pallas as pl
from jax.experimental.pallas import tpu as pltpu

_TM = 512
_VMEM = 100 * 1024 * 1024


def _cdiv(a, b):
    return (a + b - 1) // b


def _interior_mask(g, geom):
    r = jax.lax.rem(g, geom["img_p"])
    w = jax.lax.rem(r, geom["Wp"])
    ok = ((r >= geom["Wp"]) & (r < (geom["H"] + 1) * geom["Wp"])
          & (w >= 1) & (w <= geom["W"]))
    if "M" in geom:
        ok &= g < geom["M"]
    return ok


def _rows_iota(n, base):
    return jax.lax.broadcasted_iota(jnp.int32, (n, 1), 0) + base


def _mm_body(*refs, relu, has_res, geom, tm):
    if has_res:
        x_ref, w_ref, s_ref, b_ref, r_ref, o_ref = refs
    else:
        x_ref, w_ref, s_ref, b_ref, o_ref = refs
    y = jnp.dot(x_ref[...], w_ref[...], preferred_element_type=jnp.float32)
    y = y * s_ref[...] + b_ref[...]
    if has_res:
        y = y + r_ref[...].astype(jnp.float32)
    if relu:
        y = jnp.maximum(y, 0.0)
    if geom is not None:
        g = _rows_iota(y.shape[0], pl.program_id(0) * tm)
        y = jnp.where(_interior_mask(g, geom), y, 0.0)
    o_ref[...] = y.astype(o_ref.dtype)


def _mm(x, w, s, b, relu, res=None, out_dtype=jnp.bfloat16, geom=None):
    M, K = x.shape
    N = w.shape[1]
    tm = min(_TM, M)
    tn = min(N, 512)
    grid = (_cdiv(M, tm), N // tn)
    in_specs = [
        pl.BlockSpec((tm, K), lambda i, j: (i, 0)),
        pl.BlockSpec((K, tn), lambda i, j: (0, j)),
        pl.BlockSpec((1, tn), lambda i, j: (0, j)),
        pl.BlockSpec((1, tn), lambda i, j: (0, j)),
    ]
    args = [x.astype(jnp.bfloat16), w, s, b]
    if res is not None:
        in_specs.append(pl.BlockSpec((tm, tn), lambda i, j: (i, j)))
        args.append(res.astype(jnp.bfloat16))
    return pl.pallas_call(
        functools.partial(_mm_body, relu=relu, has_res=res is not None,
                          geom=geom, tm=tm),
        out_shape=jax.ShapeDtypeStruct((M, N), out_dtype),
        grid=grid,
        in_specs=in_specs,
        out_specs=pl.BlockSpec((tm, tn), lambda i, j: (i, j)),
        compiler_params=pltpu.CompilerParams(
            dimension_semantics=("parallel", "parallel"),
            vmem_limit_bytes=_VMEM),
    )(*args)


def _chain_body(*refs, geom, nblocks, has_down):
    pv, cu, nx = refs[:3]
    out = refs[-1]
    tm = cu.shape[0]
    hal = geom["hal"]
    base0 = pl.program_id(0) * tm
    win = jnp.concatenate([pv[...], cu[...], nx[...]], axis=0)
    rin0 = nblocks * hal
    cur = win[256 - rin0:256 + tm + rin0, :]
    x0 = cur
    idx = 3
    for j in range(nblocks):
        rin = (nblocks - j) * hal
        rout = rin - hal
        w1, s1, b1, w2, s2, b2, w3, s3, b3 = refs[idx:idx + 9]
        idx += 9
        C = w1.shape[1]
        t1 = jnp.dot(cur, w1[...], preferred_element_type=jnp.float32)
        t1 = jnp.maximum(t1 * s1[...] + b1[...], 0.0)
        t1 = jnp.where(
            _interior_mask(_rows_iota(tm + 2 * rin, base0 - rin), geom),
            t1, 0.0).astype(jnp.bfloat16)
        acc = None
        for dy in range(3):
            for dx in range(3):
                d = dy * geom["Wp"] + dx - hal
                t = dy * 3 + dx
                p = jnp.dot(t1[hal + d:hal + d + tm + 2 * rout, :],
                            w2[t * C:(t + 1) * C, :],
                            preferred_element_type=jnp.float32)
                acc = p if acc is None else acc + p
        ok = _interior_mask(_rows_iota(tm + 2 * rout, base0 - rout), geom)
        t2 = jnp.where(ok, jnp.maximum(acc * s2[...] + b2[...], 0.0),
                       0.0).astype(jnp.bfloat16)
        y = jnp.dot(t2, w3[...], preferred_element_type=jnp.float32)
        y = y * s3[...] + b3[...]
        if j == 0 and has_down:
            wd, sd, bd = refs[-4], refs[-3], refs[-2]
            idn = jnp.dot(x0[hal:hal + tm + 2 * rout, :], wd[...],
                          preferred_element_type=jnp.float32)
            idn = idn * sd[...] + bd[...]
        else:
            idn = cur[hal:hal + tm + 2 * rout, :].astype(jnp.float32)
        y = jnp.maximum(y + idn, 0.0)
        cur = jnp.where(ok, y, 0.0).astype(jnp.bfloat16)
    out[...] = cur


def _chain(x, geom, blocks, down=None):
    M, Cin = x.shape
    C4 = blocks[0][6].shape[1]
    tm = _TM
    nh = M // 256
    full = lambda a: pl.BlockSpec(a.shape, lambda i: (0, 0))
    in_specs = [
        pl.BlockSpec((256, Cin), lambda i: (jnp.maximum(i * 2 - 1, 0), 0)),
        pl.BlockSpec((tm, Cin), lambda i: (i, 0)),
        pl.BlockSpec((256, Cin), lambda i: (jnp.minimum(i * 2 + 2, nh - 1),
                                            0)),
    ]
    args = [x, x, x]
    for blk in blocks:
        args += list(blk)
        in_specs += [full(a) for a in blk]
    if down is not None:
        args += list(down)
        in_specs += [full(a) for a in down]
    return pl.pallas_call(
        functools.partial(_chain_body, geom=geom, nblocks=len(blocks),
                          has_down=down is not None),
        out_shape=jax.ShapeDtypeStruct((M, C4), jnp.bfloat16),
        grid=(M // tm,),
        in_specs=in_specs,
        out_specs=pl.BlockSpec((tm, C4), lambda i: (i, 0)),
        compiler_params=pltpu.CompilerParams(
            dimension_semantics=("parallel",),
            vmem_limit_bytes=_VMEM),
    )(*args)


def _pool_body(*refs):
    acc = refs[0][...]
    for r in refs[1:-1]:
        acc = jnp.maximum(acc, r[...])
    refs[-1][...] = acc


def _stem_body(pv, cu, nx, w_ref, s_ref, b_ref, o_ref, *, tm, W):
    hw = jnp.concatenate([pv[7:, :], cu[...], nx[:1, :]], axis=0)
    y = jnp.dot(hw, w_ref[...], preferred_element_type=jnp.float32)
    y = jnp.maximum(y * s_ref[...] + b_ref[...], 0.0)
    wcol = jax.lax.rem(_rows_iota(tm, pl.program_id(0) * tm), W)
    left = jnp.where(wcol >= 1, y[0:tm, :], -jnp.inf)
    right = jnp.where(wcol <= W - 2, y[2:tm + 2, :], -jnp.inf)
    o_ref[...] = jnp.maximum(jnp.maximum(y[1:tm + 1, :], left),
                             right).astype(o_ref.dtype)


def _gap_proj_body(x_ref, w_ref, s_ref, b_ref, o_ref, *, hw):
    f = jnp.sum(x_ref[...].astype(jnp.float32), axis=1) * (1.0 / hw)
    y = jnp.dot(f.astype(jnp.bfloat16), w_ref[...],
                preferred_element_type=jnp.float32)
    o_ref[...] = y * s_ref[...] + b_ref[...]


def _geom(H, W, B):
    Hp, Wp = H + 2, W + 2
    img = Hp * Wp
    img_p = _cdiv(img, 16) * 16
    return {"H": H, "W": W, "Wp": Wp, "img": img, "img_p": img_p,
            "hal": Wp + 1, "M": B * img_p}


def _to_layout(x, geom):
    B, H, W, C = x.shape
    xp = jnp.pad(x, ((0, 0), (1, 1), (1, 1), (0, 0)))
    xp = xp.reshape(B, geom["img"], C)
    xp = jnp.pad(xp, ((0, 0), (0, geom["img_p"] - geom["img"]), (0, 0)))
    return xp.reshape(B * geom["img_p"], C)


def _from_layout(x, geom, B):
    C = x.shape[1]
    return (x.reshape(B, geom["img_p"], C)[:, :geom["img"], :]
            .reshape(B, geom["H"] + 2, geom["Wp"], C))


def _im2col_s2(xpad, Ho, Wo, k=3):
    cols = [xpad[:, dy:dy + 2 * Ho - 1:2, dx:dx + 2 * Wo - 1:2, :]
            for dy in range(k) for dx in range(k)]
    B = xpad.shape[0]
    return jnp.concatenate(cols, axis=-1).reshape(B * Ho * Wo, -1)


def _block_s2(x, gin, gout, B, p):
    (w1, s1, b1, w2, s2, b2, w3, s3, b3, wd, sd, bd) = p
    Ho, Wo = gout["H"], gout["W"]
    t1 = _mm(x, w1, s1, b1, relu=True, geom=gin)
    t1p = _from_layout(t1, gin, B)
    rows = _im2col_s2(t1p, Ho, Wo)
    t2 = _mm(rows, w2, s2, b2, relu=True)
    xc = _from_layout(x, gin, B)[:, 1:2 * Ho:2, 1:2 * Wo:2, :]
    idn = _mm(xc.reshape(B * Ho * Wo, -1), wd, sd, bd, relu=False)
    y = _mm(t2, w3, s3, b3, relu=True, res=idn)
    return _to_layout(y.reshape(B, Ho, Wo, -1), gout)


def kernel(images, conv1, bn1_s, bn1_b, L1B0_conv1, L1B0_conv2, L1B0_conv3, L1B0_s1, L1B0_b1, L1B0_s2, L1B0_b2, L1B0_s3, L1B0_b3, L1B0_down, L1B0_sd, L1B0_bd, L1B1_conv1, L1B1_conv2, L1B1_conv3, L1B1_s1, L1B1_b1, L1B1_s2, L1B1_b2, L1B1_s3, L1B1_b3, L1B2_conv1, L1B2_conv2, L1B2_conv3, L1B2_s1, L1B2_b1, L1B2_s2, L1B2_b2, L1B2_s3, L1B2_b3, L2B0_conv1, L2B0_conv2, L2B0_conv3, L2B0_s1, L2B0_b1, L2B0_s2, L2B0_b2, L2B0_s3, L2B0_b3, L2B0_down, L2B0_sd, L2B0_bd, L2B1_conv1, L2B1_conv2, L2B1_conv3, L2B1_s1, L2B1_b1, L2B1_s2, L2B1_b2, L2B1_s3, L2B1_b3, L2B2_conv1, L2B2_conv2, L2B2_conv3, L2B2_s1, L2B2_b1, L2B2_s2, L2B2_b2, L2B2_s3, L2B2_b3, L2B3_conv1, L2B3_conv2, L2B3_conv3, L2B3_s1, L2B3_b1, L2B3_s2, L2B3_b2, L2B3_s3, L2B3_b3, L3B0_conv1, L3B0_conv2, L3B0_conv3, L3B0_s1, L3B0_b1, L3B0_s2, L3B0_b2, L3B0_s3, L3B0_b3, L3B0_down, L3B0_sd, L3B0_bd, L3B1_conv1, L3B1_conv2, L3B1_conv3, L3B1_s1, L3B1_b1, L3B1_s2, L3B1_b2, L3B1_s3, L3B1_b3, L3B2_conv1, L3B2_conv2, L3B2_conv3, L3B2_s1, L3B2_b1, L3B2_s2, L3B2_b2, L3B2_s3, L3B2_b3, L3B3_conv1, L3B3_conv2, L3B3_conv3, L3B3_s1, L3B3_b1, L3B3_s2, L3B3_b2, L3B3_s3, L3B3_b3, L3B4_conv1, L3B4_conv2, L3B4_conv3, L3B4_s1, L3B4_b1, L3B4_s2, L3B4_b2, L3B4_s3, L3B4_b3, L3B5_conv1, L3B5_conv2, L3B5_conv3, L3B5_s1, L3B5_b1, L3B5_s2, L3B5_b2, L3B5_s3, L3B5_b3, L4B0_conv1, L4B0_conv2, L4B0_conv3, L4B0_s1, L4B0_b1, L4B0_s2, L4B0_b2, L4B0_s3, L4B0_b3, L4B0_down, L4B0_sd, L4B0_bd, L4B1_conv1, L4B1_conv2, L4B1_conv3, L4B1_s1, L4B1_b1, L4B1_s2, L4B1_b2, L4B1_s3, L4B1_b3, L4B2_conv1, L4B2_conv2, L4B2_conv3, L4B2_s1, L4B2_b1, L4B2_s2, L4B2_b2, L4B2_s3, L4B2_b3, proj_w, proj_s, proj_b):
    B = images.shape[0]
    g1, g2, g3, g4 = (_geom(56, 56, B), _geom(28, 28, B), _geom(14, 14, B),
                      _geom(7, 7, B))

    x = jnp.transpose(images, (0, 2, 3, 1)).astype(jnp.bfloat16)
    xp = jnp.pad(x, ((0, 0), (3, 3), (3, 3), (0, 0)))
    cols = [xp[:, dy:dy + 223:2, dx:dx + 223:2, :]
            for dy in range(7) for dx in range(7)]
    cols.append(jnp.zeros((B, 112, 112, 160 - 147), jnp.bfloat16))
    rows = jnp.concatenate(cols, axis=-1).reshape(B * 112 * 112, 160)
    M0 = B * 112 * 112
    nh0 = M0 // 8
    wmax = pl.pallas_call(
        functools.partial(_stem_body, tm=_TM, W=112),
        out_shape=jax.ShapeDtypeStruct((M0, 128), jnp.bfloat16),
        grid=(M0 // _TM,),
        in_specs=[
            pl.BlockSpec((8, 160), lambda i: (jnp.maximum(i * 64 - 1, 0), 0)),
            pl.BlockSpec((_TM, 160), lambda i: (i, 0)),
            pl.BlockSpec((8, 160), lambda i: (jnp.minimum(i * 64 + 64,
                                                          nh0 - 1), 0)),
            pl.BlockSpec((160, 128), lambda i: (0, 0)),
            pl.BlockSpec((1, 128), lambda i: (0, 0)),
            pl.BlockSpec((1, 128), lambda i: (0, 0)),
        ],
        out_specs=pl.BlockSpec((_TM, 128), lambda i: (i, 0)),
        compiler_params=pltpu.CompilerParams(
            dimension_semantics=("parallel",), vmem_limit_bytes=_VMEM),
    )(rows, rows, rows, conv1[:160, :], bn1_s, bn1_b)

    yp = jnp.pad(wmax.reshape(B, 112, 112, 128), ((0, 0), (1, 1), (0, 0),
                                                  (0, 0)),
                 constant_values=-jnp.inf)
    taps = [yp[:, dy:dy + 111:2, 0:112:2, :].reshape(B * 56 * 56, 128)
            for dy in range(3)]
    M1 = B * 56 * 56
    pooled = pl.pallas_call(
        _pool_body,
        out_shape=jax.ShapeDtypeStruct((M1, 128), jnp.bfloat16),
        grid=(M1 // _TM,),
        in_specs=[pl.BlockSpec((_TM, 128), lambda i: (i, 0))] * 3,
        out_specs=pl.BlockSpec((_TM, 128), lambda i: (i, 0)),
        compiler_params=pltpu.CompilerParams(
            dimension_semantics=("parallel",), vmem_limit_bytes=_VMEM),
    )(*taps)
    x = _to_layout(pooled.reshape(B, 56, 56, 128), g1)

    x = _chain(x, g1,
               [(L1B0_conv1, L1B0_s1, L1B0_b1, L1B0_conv2, L1B0_s2, L1B0_b2,
                 L1B0_conv3, L1B0_s3, L1B0_b3),
                (L1B1_conv1, L1B1_s1, L1B1_b1, L1B1_conv2, L1B1_s2, L1B1_b2,
                 L1B1_conv3, L1B1_s3, L1B1_b3),
                (L1B2_conv1, L1B2_s1, L1B2_b1, L1B2_conv2, L1B2_s2, L1B2_b2,
                 L1B2_conv3, L1B2_s3, L1B2_b3)],
               down=(L1B0_down, L1B0_sd, L1B0_bd))

    x = _block_s2(x, g1, g2, B, (L2B0_conv1, L2B0_s1, L2B0_b1, L2B0_conv2,
                                 L2B0_s2, L2B0_b2, L2B0_conv3, L2B0_s3,
                                 L2B0_b3, L2B0_down, L2B0_sd, L2B0_bd))
    x = _chain(x, g2,
               [(L2B1_conv1, L2B1_s1, L2B1_b1, L2B1_conv2, L2B1_s2, L2B1_b2,
                 L2B1_conv3, L2B1_s3, L2B1_b3),
                (L2B2_conv1, L2B2_s1, L2B2_b1, L2B2_conv2, L2B2_s2, L2B2_b2,
                 L2B2_conv3, L2B2_s3, L2B2_b3),
                (L2B3_conv1, L2B3_s1, L2B3_b1, L2B3_conv2, L2B3_s2, L2B3_b2,
                 L2B3_conv3, L2B3_s3, L2B3_b3)])

    x = _block_s2(x, g2, g3, B, (L3B0_conv1, L3B0_s1, L3B0_b1, L3B0_conv2,
                                 L3B0_s2, L3B0_b2, L3B0_conv3, L3B0_s3,
                                 L3B0_b3, L3B0_down, L3B0_sd, L3B0_bd))
    x = _chain(x, g3,
               [(L3B1_conv1, L3B1_s1, L3B1_b1, L3B1_conv2, L3B1_s2, L3B1_b2,
                 L3B1_conv3, L3B1_s3, L3B1_b3),
                (L3B2_conv1, L3B2_s1, L3B2_b1, L3B2_conv2, L3B2_s2, L3B2_b2,
                 L3B2_conv3, L3B2_s3, L3B2_b3),
                (L3B3_conv1, L3B3_s1, L3B3_b1, L3B3_conv2, L3B3_s2, L3B3_b2,
                 L3B3_conv3, L3B3_s3, L3B3_b3),
                (L3B4_conv1, L3B4_s1, L3B4_b1, L3B4_conv2, L3B4_s2, L3B4_b2,
                 L3B4_conv3, L3B4_s3, L3B4_b3),
                (L3B5_conv1, L3B5_s1, L3B5_b1, L3B5_conv2, L3B5_s2, L3B5_b2,
                 L3B5_conv3, L3B5_s3, L3B5_b3)])

    x = _block_s2(x, g3, g4, B, (L4B0_conv1, L4B0_s1, L4B0_b1, L4B0_conv2,
                                 L4B0_s2, L4B0_b2, L4B0_conv3, L4B0_s3,
                                 L4B0_b3, L4B0_down, L4B0_sd, L4B0_bd))
    x = _chain(x, g4,
               [(L4B1_conv1, L4B1_s1, L4B1_b1, L4B1_conv2, L4B1_s2, L4B1_b2,
                 L4B1_conv3, L4B1_s3, L4B1_b3),
                (L4B2_conv1, L4B2_s1, L4B2_b1, L4B2_conv2, L4B2_s2, L4B2_b2,
                 L4B2_conv3, L4B2_s3, L4B2_b3)])

    x3 = x.reshape(B, g4["img_p"], 2048)
    out = pl.pallas_call(
        functools.partial(_gap_proj_body, hw=49.0),
        out_shape=jax.ShapeDtypeStruct((B, 512), jnp.float32),
        compiler_params=pltpu.CompilerParams(vmem_limit_bytes=_VMEM),
    )(x3, proj_w, proj_s, proj_b)
    return out.reshape(B, 1, 512)


# speedup vs baseline: 1.0124x; 1.0124x over previous
import functools

import jax
import jax.numpy as jnp
from jax.experimental import pallas as pl
from jax.experimental.pallas import tpu as pltpu

_TM = 512
_VMEM = 100 * 1024 * 1024


def _cdiv(a, b):
    return (a + b - 1) // b


def _interior_mask(g, geom):
    r = jax.lax.rem(g, geom["img_p"])
    w = jax.lax.rem(r, geom["Wp"])
    ok = ((r >= geom["Wp"]) & (r < (geom["H"] + 1) * geom["Wp"])
          & (w >= 1) & (w <= geom["W"]))
    if "M" in geom:
        ok &= g < geom["M"]
    return ok


def _rows_iota(n, base):
    return jax.lax.broadcasted_iota(jnp.int32, (n, 1), 0) + base


def _mm_body(*refs, relu, has_res, geom, tm):
    if has_res:
        x_ref, w_ref, s_ref, b_ref, r_ref, o_ref = refs
    else:
        x_ref, w_ref, s_ref, b_ref, o_ref = refs
    y = jnp.dot(x_ref[...], w_ref[...], preferred_element_type=jnp.float32)
    y = y * s_ref[...] + b_ref[...]
    if has_res:
        y = y + r_ref[...].astype(jnp.float32)
    if relu:
        y = jnp.maximum(y, 0.0)
    if geom is not None:
        g = _rows_iota(y.shape[0], pl.program_id(0) * tm)
        y = jnp.where(_interior_mask(g, geom), y, 0.0)
    o_ref[...] = y.astype(o_ref.dtype)


def _mm(x, w, s, b, relu, res=None, out_dtype=jnp.bfloat16, geom=None):
    M, K = x.shape
    N = w.shape[1]
    tm = min(_TM, M)
    tn = min(N, 512)
    grid = (_cdiv(M, tm), N // tn)
    in_specs = [
        pl.BlockSpec((tm, K), lambda i, j: (i, 0)),
        pl.BlockSpec((K, tn), lambda i, j: (0, j)),
        pl.BlockSpec((1, tn), lambda i, j: (0, j)),
        pl.BlockSpec((1, tn), lambda i, j: (0, j)),
    ]
    args = [x.astype(jnp.bfloat16), w, s, b]
    if res is not None:
        in_specs.append(pl.BlockSpec((tm, tn), lambda i, j: (i, j)))
        args.append(res.astype(jnp.bfloat16))
    return pl.pallas_call(
        functools.partial(_mm_body, relu=relu, has_res=res is not None,
                          geom=geom, tm=tm),
        out_shape=jax.ShapeDtypeStruct((M, N), out_dtype),
        grid=grid,
        in_specs=in_specs,
        out_specs=pl.BlockSpec((tm, tn), lambda i, j: (i, j)),
        compiler_params=pltpu.CompilerParams(
            dimension_semantics=("parallel", "parallel"),
            vmem_limit_bytes=_VMEM),
    )(*args)


def _chain_body(*refs, geom, nblocks, has_down):
    pv, cu, nx = refs[:3]
    out = refs[-1]
    tm = cu.shape[0]
    hal = geom["hal"]
    base0 = pl.program_id(0) * tm
    win = jnp.concatenate([pv[...], cu[...], nx[...]], axis=0)
    rin0 = nblocks * hal
    cur = win[256 - rin0:256 + tm + rin0, :]
    x0 = cur
    idx = 3
    for j in range(nblocks):
        rin = (nblocks - j) * hal
        rout = rin - hal
        w1, s1, b1, w2, s2, b2, w3, s3, b3 = refs[idx:idx + 9]
        idx += 9
        C = w1.shape[1]
        t1 = jnp.dot(cur, w1[...], preferred_element_type=jnp.float32)
        t1 = jnp.maximum(t1 * s1[...] + b1[...], 0.0)
        t1 = jnp.where(
            _interior_mask(_rows_iota(tm + 2 * rin, base0 - rin), geom),
            t1, 0.0).astype(jnp.bfloat16)
        acc = None
        for dy in range(3):
            for dx in range(3):
                d = dy * geom["Wp"] + dx - hal
                t = dy * 3 + dx
                p = jnp.dot(t1[hal + d:hal + d + tm + 2 * rout, :],
                            w2[t * C:(t + 1) * C, :],
                            preferred_element_type=jnp.float32)
                acc = p if acc is None else acc + p
        ok = _interior_mask(_rows_iota(tm + 2 * rout, base0 - rout), geom)
        t2 = jnp.where(ok, jnp.maximum(acc * s2[...] + b2[...], 0.0),
                       0.0).astype(jnp.bfloat16)
        y = jnp.dot(t2, w3[...], preferred_element_type=jnp.float32)
        y = y * s3[...] + b3[...]
        if j == 0 and has_down:
            wd, sd, bd = refs[-4], refs[-3], refs[-2]
            idn = jnp.dot(x0[hal:hal + tm + 2 * rout, :], wd[...],
                          preferred_element_type=jnp.float32)
            idn = idn * sd[...] + bd[...]
        else:
            idn = cur[hal:hal + tm + 2 * rout, :].astype(jnp.float32)
        y = jnp.maximum(y + idn, 0.0)
        cur = jnp.where(ok, y, 0.0).astype(jnp.bfloat16)
    out[...] = cur


def _chain(x, geom, blocks, down=None):
    M, Cin = x.shape
    C4 = blocks[0][6].shape[1]
    tm = _TM
    nh = M // 256
    full = lambda a: pl.BlockSpec(a.shape, lambda i: (0, 0))
    in_specs = [
        pl.BlockSpec((256, Cin), lambda i: (jnp.maximum(i * 2 - 1, 0), 0)),
        pl.BlockSpec((tm, Cin), lambda i: (i, 0)),
        pl.BlockSpec((256, Cin), lambda i: (jnp.minimum(i * 2 + 2, nh - 1),
                                            0)),
    ]
    args = [x, x, x]
    for blk in blocks:
        args += list(blk)
        in_specs += [full(a) for a in blk]
    if down is not None:
        args += list(down)
        in_specs += [full(a) for a in down]
    return pl.pallas_call(
        functools.partial(_chain_body, geom=geom, nblocks=len(blocks),
                          has_down=down is not None),
        out_shape=jax.ShapeDtypeStruct((M, C4), jnp.bfloat16),
        grid=(M // tm,),
        in_specs=in_specs,
        out_specs=pl.BlockSpec((tm, C4), lambda i: (i, 0)),
        compiler_params=pltpu.CompilerParams(
            dimension_semantics=("parallel",),
            vmem_limit_bytes=_VMEM),
    )(*args)


def _pool_body(*refs):
    acc = refs[0][...]
    for r in refs[1:-1]:
        acc = jnp.maximum(acc, r[...])
    refs[-1][...] = acc


def _stem_body(pv, cu, nx, w_ref, s_ref, b_ref, o_ref, *, tm, W):
    hw = jnp.concatenate([pv[7:, :], cu[...], nx[:1, :]], axis=0)
    y = jnp.dot(hw, w_ref[...], preferred_element_type=jnp.float32)
    y = jnp.maximum(y * s_ref[...] + b_ref[...], 0.0)
    wcol = jax.lax.rem(_rows_iota(tm, pl.program_id(0) * tm), W)
    left = jnp.where(wcol >= 1, y[0:tm, :], -jnp.inf)
    right = jnp.where(wcol <= W - 2, y[2:tm + 2, :], -jnp.inf)
    o_ref[...] = jnp.maximum(jnp.maximum(y[1:tm + 1, :], left),
                             right).astype(o_ref.dtype)


def _gap_proj_body(x_ref, w_ref, s_ref, b_ref, o_ref, *, hw):
    f = jnp.sum(x_ref[...].astype(jnp.float32), axis=1) * (1.0 / hw)
    y = jnp.dot(f.astype(jnp.bfloat16), w_ref[...],
                preferred_element_type=jnp.float32)
    o_ref[...] = y * s_ref[...] + b_ref[...]


def _geom(H, W, B):
    Hp, Wp = H + 2, W + 2
    img = Hp * Wp
    img_p = _cdiv(img, 16) * 16
    return {"H": H, "W": W, "Wp": Wp, "img": img, "img_p": img_p,
            "hal": Wp + 1, "M": B * img_p}


def _to_layout(x, geom):
    B, H, W, C = x.shape
    xp = jnp.pad(x, ((0, 0), (1, 1), (1, 1), (0, 0)))
    xp = xp.reshape(B, geom["img"], C)
    xp = jnp.pad(xp, ((0, 0), (0, geom["img_p"] - geom["img"]), (0, 0)))
    return xp.reshape(B * geom["img_p"], C)


def _from_layout(x, geom, B):
    C = x.shape[1]
    return (x.reshape(B, geom["img_p"], C)[:, :geom["img"], :]
            .reshape(B, geom["H"] + 2, geom["Wp"], C))


def _im2col_s2(xpad, Ho, Wo, k=3):
    cols = [xpad[:, dy:dy + 2 * Ho - 1:2, dx:dx + 2 * Wo - 1:2, :]
            for dy in range(k) for dx in range(k)]
    B = xpad.shape[0]
    return jnp.concatenate(cols, axis=-1).reshape(B * Ho * Wo, -1)


def _block_s2(x, gin, gout, B, p):
    (w1, s1, b1, w2, s2, b2, w3, s3, b3, wd, sd, bd) = p
    Ho, Wo = gout["H"], gout["W"]
    t1 = _mm(x, w1, s1, b1, relu=True, geom=gin)
    t1p = _from_layout(t1, gin, B)
    rows = _im2col_s2(t1p, Ho, Wo)
    t2 = _mm(rows, w2, s2, b2, relu=True)
    xc = _from_layout(x, gin, B)[:, 1:2 * Ho:2, 1:2 * Wo:2, :]
    idn = _mm(xc.reshape(B * Ho * Wo, -1), wd, sd, bd, relu=False)
    y = _mm(t2, w3, s3, b3, relu=True, res=idn)
    return _to_layout(y.reshape(B, Ho, Wo, -1), gout)


def kernel(images, conv1, bn1_s, bn1_b, L1B0_conv1, L1B0_conv2, L1B0_conv3, L1B0_s1, L1B0_b1, L1B0_s2, L1B0_b2, L1B0_s3, L1B0_b3, L1B0_down, L1B0_sd, L1B0_bd, L1B1_conv1, L1B1_conv2, L1B1_conv3, L1B1_s1, L1B1_b1, L1B1_s2, L1B1_b2, L1B1_s3, L1B1_b3, L1B2_conv1, L1B2_conv2, L1B2_conv3, L1B2_s1, L1B2_b1, L1B2_s2, L1B2_b2, L1B2_s3, L1B2_b3, L2B0_conv1, L2B0_conv2, L2B0_conv3, L2B0_s1, L2B0_b1, L2B0_s2, L2B0_b2, L2B0_s3, L2B0_b3, L2B0_down, L2B0_sd, L2B0_bd, L2B1_conv1, L2B1_conv2, L2B1_conv3, L2B1_s1, L2B1_b1, L2B1_s2, L2B1_b2, L2B1_s3, L2B1_b3, L2B2_conv1, L2B2_conv2, L2B2_conv3, L2B2_s1, L2B2_b1, L2B2_s2, L2B2_b2, L2B2_s3, L2B2_b3, L2B3_conv1, L2B3_conv2, L2B3_conv3, L2B3_s1, L2B3_b1, L2B3_s2, L2B3_b2, L2B3_s3, L2B3_b3, L3B0_conv1, L3B0_conv2, L3B0_conv3, L3B0_s1, L3B0_b1, L3B0_s2, L3B0_b2, L3B0_s3, L3B0_b3, L3B0_down, L3B0_sd, L3B0_bd, L3B1_conv1, L3B1_conv2, L3B1_conv3, L3B1_s1, L3B1_b1, L3B1_s2, L3B1_b2, L3B1_s3, L3B1_b3, L3B2_conv1, L3B2_conv2, L3B2_conv3, L3B2_s1, L3B2_b1, L3B2_s2, L3B2_b2, L3B2_s3, L3B2_b3, L3B3_conv1, L3B3_conv2, L3B3_conv3, L3B3_s1, L3B3_b1, L3B3_s2, L3B3_b2, L3B3_s3, L3B3_b3, L3B4_conv1, L3B4_conv2, L3B4_conv3, L3B4_s1, L3B4_b1, L3B4_s2, L3B4_b2, L3B4_s3, L3B4_b3, L3B5_conv1, L3B5_conv2, L3B5_conv3, L3B5_s1, L3B5_b1, L3B5_s2, L3B5_b2, L3B5_s3, L3B5_b3, L4B0_conv1, L4B0_conv2, L4B0_conv3, L4B0_s1, L4B0_b1, L4B0_s2, L4B0_b2, L4B0_s3, L4B0_b3, L4B0_down, L4B0_sd, L4B0_bd, L4B1_conv1, L4B1_conv2, L4B1_conv3, L4B1_s1, L4B1_b1, L4B1_s2, L4B1_b2, L4B1_s3, L4B1_b3, L4B2_conv1, L4B2_conv2, L4B2_conv3, L4B2_s1, L4B2_b1, L4B2_s2, L4B2_b2, L4B2_s3, L4B2_b3, proj_w, proj_s, proj_b):
    B = images.shape[0]
    g1, g2, g3, g4 = (_geom(56, 56, B), _geom(28, 28, B), _geom(14, 14, B),
                      _geom(7, 7, B))

    x = jnp.transpose(images, (0, 2, 3, 1)).astype(jnp.bfloat16)
    xp = jnp.pad(x, ((0, 0), (3, 3), (3, 3), (0, 0)))
    cols = [xp[:, dy:dy + 223:2, dx:dx + 223:2, :]
            for dy in range(7) for dx in range(7)]
    cols.append(jnp.zeros((B, 112, 112, 160 - 147), jnp.bfloat16))
    rows = jnp.concatenate(cols, axis=-1).reshape(B * 112 * 112, 160)
    M0 = B * 112 * 112
    nh0 = M0 // 8
    wmax = pl.pallas_call(
        functools.partial(_stem_body, tm=_TM, W=112),
        out_shape=jax.ShapeDtypeStruct((M0, 128), jnp.bfloat16),
        grid=(M0 // _TM,),
        in_specs=[
            pl.BlockSpec((8, 160), lambda i: (jnp.maximum(i * 64 - 1, 0), 0)),
            pl.BlockSpec((_TM, 160), lambda i: (i, 0)),
            pl.BlockSpec((8, 160), lambda i: (jnp.minimum(i * 64 + 64,
                                                          nh0 - 1), 0)),
            pl.BlockSpec((160, 128), lambda i: (0, 0)),
            pl.BlockSpec((1, 128), lambda i: (0, 0)),
            pl.BlockSpec((1, 128), lambda i: (0, 0)),
        ],
        out_specs=pl.BlockSpec((_TM, 128), lambda i: (i, 0)),
        compiler_params=pltpu.CompilerParams(
            dimension_semantics=("parallel",), vmem_limit_bytes=_VMEM),
    )(rows, rows, rows, conv1[:160, :], bn1_s, bn1_b)

    yp = jnp.pad(wmax.reshape(B, 112, 112, 128), ((0, 0), (1, 1), (0, 0),
                                                  (0, 0)),
                 constant_values=-jnp.inf)
    taps = [yp[:, dy:dy + 111:2, 0:112:2, :].reshape(B * 56 * 56, 128)
            for dy in range(3)]
    M1 = B * 56 * 56
    pooled = pl.pallas_call(
        _pool_body,
        out_shape=jax.ShapeDtypeStruct((M1, 128), jnp.bfloat16),
        grid=(M1 // _TM,),
        in_specs=[pl.BlockSpec((_TM, 128), lambda i: (i, 0))] * 3,
        out_specs=pl.BlockSpec((_TM, 128), lambda i: (i, 0)),
        compiler_params=pltpu.CompilerParams(
            dimension_semantics=("parallel",), vmem_limit_bytes=_VMEM),
    )(*taps)
    x = _to_layout(pooled.reshape(B, 56, 56, 128), g1)

    x = _chain(x, g1,
               [(L1B0_conv1, L1B0_s1, L1B0_b1, L1B0_conv2, L1B0_s2, L1B0_b2,
                 L1B0_conv3, L1B0_s3, L1B0_b3)],
               down=(L1B0_down, L1B0_sd, L1B0_bd))
    x = _chain(x, g1,
               [(L1B1_conv1, L1B1_s1, L1B1_b1, L1B1_conv2, L1B1_s2, L1B1_b2,
                 L1B1_conv3, L1B1_s3, L1B1_b3)])
    x = _chain(x, g1,
               [(L1B2_conv1, L1B2_s1, L1B2_b1, L1B2_conv2, L1B2_s2, L1B2_b2,
                 L1B2_conv3, L1B2_s3, L1B2_b3)])

    x = _block_s2(x, g1, g2, B, (L2B0_conv1, L2B0_s1, L2B0_b1, L2B0_conv2,
                                 L2B0_s2, L2B0_b2, L2B0_conv3, L2B0_s3,
                                 L2B0_b3, L2B0_down, L2B0_sd, L2B0_bd))
    x = _chain(x, g2,
               [(L2B1_conv1, L2B1_s1, L2B1_b1, L2B1_conv2, L2B1_s2, L2B1_b2,
                 L2B1_conv3, L2B1_s3, L2B1_b3),
                (L2B2_conv1, L2B2_s1, L2B2_b1, L2B2_conv2, L2B2_s2, L2B2_b2,
                 L2B2_conv3, L2B2_s3, L2B2_b3),
                (L2B3_conv1, L2B3_s1, L2B3_b1, L2B3_conv2, L2B3_s2, L2B3_b2,
                 L2B3_conv3, L2B3_s3, L2B3_b3)])

    x = _block_s2(x, g2, g3, B, (L3B0_conv1, L3B0_s1, L3B0_b1, L3B0_conv2,
                                 L3B0_s2, L3B0_b2, L3B0_conv3, L3B0_s3,
                                 L3B0_b3, L3B0_down, L3B0_sd, L3B0_bd))
    x = _chain(x, g3,
               [(L3B1_conv1, L3B1_s1, L3B1_b1, L3B1_conv2, L3B1_s2, L3B1_b2,
                 L3B1_conv3, L3B1_s3, L3B1_b3),
                (L3B2_conv1, L3B2_s1, L3B2_b1, L3B2_conv2, L3B2_s2, L3B2_b2,
                 L3B2_conv3, L3B2_s3, L3B2_b3),
                (L3B3_conv1, L3B3_s1, L3B3_b1, L3B3_conv2, L3B3_s2, L3B3_b2,
                 L3B3_conv3, L3B3_s3, L3B3_b3),
                (L3B4_conv1, L3B4_s1, L3B4_b1, L3B4_conv2, L3B4_s2, L3B4_b2,
                 L3B4_conv3, L3B4_s3, L3B4_b3),
                (L3B5_conv1, L3B5_s1, L3B5_b1, L3B5_conv2, L3B5_s2, L3B5_b2,
                 L3B5_conv3, L3B5_s3, L3B5_b3)])

    x = _block_s2(x, g3, g4, B, (L4B0_conv1, L4B0_s1, L4B0_b1, L4B0_conv2,
                                 L4B0_s2, L4B0_b2, L4B0_conv3, L4B0_s3,
                                 L4B0_b3, L4B0_down, L4B0_sd, L4B0_bd))
    x = _chain(x, g4,
               [(L4B1_conv1, L4B1_s1, L4B1_b1, L4B1_conv2, L4B1_s2, L4B1_b2,
                 L4B1_conv3, L4B1_s3, L4B1_b3),
                (L4B2_conv1, L4B2_s1, L4B2_b1, L4B2_conv2, L4B2_s2, L4B2_b2,
                 L4B2_conv3, L4B2_s3, L4B2_b3)])

    x3 = x.reshape(B, g4["img_p"], 2048)
    out = pl.pallas_call(
        functools.partial(_gap_proj_body, hw=49.0),
        out_shape=jax.ShapeDtypeStruct((B, 512), jnp.float32),
        compiler_params=pltpu.CompilerParams(vmem_limit_bytes=_VMEM),
    )(x3, proj_w, proj_s, proj_b)
    return out.reshape(B, 1, 512)


# speedup vs baseline: 1.6178x; 1.5979x over previous
import functools

import jax
import jax.numpy as jnp
from jax.experimental import pallas as pl
from jax.experimental.pallas import tpu as pltpu

_TM = 512
_VMEM = 100 * 1024 * 1024


def _cdiv(a, b):
    return (a + b - 1) // b


def _interior_mask(g, geom):
    r = jax.lax.rem(g, geom["img_p"])
    w = jax.lax.rem(r, geom["Wp"])
    ok = ((r >= geom["Wp"]) & (r < (geom["H"] + 1) * geom["Wp"])
          & (w >= 1) & (w <= geom["W"]))
    if "M" in geom:
        ok &= g < geom["M"]
    return ok


def _rows_iota(n, base):
    return jax.lax.broadcasted_iota(jnp.int32, (n, 1), 0) + base


def _mm_body(*refs, relu, has_res, geom, tm):
    if has_res:
        x_ref, w_ref, s_ref, b_ref, r_ref, o_ref = refs
    else:
        x_ref, w_ref, s_ref, b_ref, o_ref = refs
    y = jnp.dot(x_ref[...], w_ref[...], preferred_element_type=jnp.float32)
    y = y * s_ref[...] + b_ref[...]
    if has_res:
        y = y + r_ref[...].astype(jnp.float32)
    if relu:
        y = jnp.maximum(y, 0.0)
    if geom is not None:
        g = _rows_iota(y.shape[0], pl.program_id(0) * tm)
        y = jnp.where(_interior_mask(g, geom), y, 0.0)
    o_ref[...] = y.astype(o_ref.dtype)


def _mm(x, w, s, b, relu, res=None, out_dtype=jnp.bfloat16, geom=None):
    M, K = x.shape
    N = w.shape[1]
    tm = min(_TM, M)
    tn = min(N, 512)
    grid = (_cdiv(M, tm), N // tn)
    in_specs = [
        pl.BlockSpec((tm, K), lambda i, j: (i, 0)),
        pl.BlockSpec((K, tn), lambda i, j: (0, j)),
        pl.BlockSpec((1, tn), lambda i, j: (0, j)),
        pl.BlockSpec((1, tn), lambda i, j: (0, j)),
    ]
    args = [x.astype(jnp.bfloat16), w, s, b]
    if res is not None:
        in_specs.append(pl.BlockSpec((tm, tn), lambda i, j: (i, j)))
        args.append(res.astype(jnp.bfloat16))
    return pl.pallas_call(
        functools.partial(_mm_body, relu=relu, has_res=res is not None,
                          geom=geom, tm=tm),
        out_shape=jax.ShapeDtypeStruct((M, N), out_dtype),
        grid=grid,
        in_specs=in_specs,
        out_specs=pl.BlockSpec((tm, tn), lambda i, j: (i, j)),
        compiler_params=pltpu.CompilerParams(
            dimension_semantics=("parallel", "parallel"),
            vmem_limit_bytes=_VMEM),
    )(*args)


def _chain_body(*refs, geom, nblocks, has_down):
    pv, cu, nx = refs[:3]
    out = refs[-1]
    tm = cu.shape[0]
    hal = geom["hal"]
    base0 = pl.program_id(0) * tm
    win = jnp.concatenate([pv[...], cu[...], nx[...]], axis=0)
    rin0 = nblocks * hal
    cur = win[256 - rin0:256 + tm + rin0, :]
    x0 = cur
    idx = 3
    for j in range(nblocks):
        rin = (nblocks - j) * hal
        rout = rin - hal
        w1, s1, b1, w2, s2, b2, w3, s3, b3 = refs[idx:idx + 9]
        idx += 9
        C = w1.shape[1]
        t1 = jnp.dot(cur, w1[...], preferred_element_type=jnp.float32)
        t1 = jnp.maximum(t1 * s1[...] + b1[...], 0.0)
        t1 = jnp.where(
            _interior_mask(_rows_iota(tm + 2 * rin, base0 - rin), geom),
            t1, 0.0).astype(jnp.bfloat16)
        acc = None
        for dy in range(3):
            for dx in range(3):
                d = dy * geom["Wp"] + dx - hal
                t = dy * 3 + dx
                p = jnp.dot(t1[hal + d:hal + d + tm + 2 * rout, :],
                            w2[t * C:(t + 1) * C, :],
                            preferred_element_type=jnp.float32)
                acc = p if acc is None else acc + p
        ok = _interior_mask(_rows_iota(tm + 2 * rout, base0 - rout), geom)
        t2 = jnp.where(ok, jnp.maximum(acc * s2[...] + b2[...], 0.0),
                       0.0).astype(jnp.bfloat16)
        y = jnp.dot(t2, w3[...], preferred_element_type=jnp.float32)
        y = y * s3[...] + b3[...]
        if j == 0 and has_down:
            wd, sd, bd = refs[-4], refs[-3], refs[-2]
            idn = jnp.dot(x0[hal:hal + tm + 2 * rout, :], wd[...],
                          preferred_element_type=jnp.float32)
            idn = idn * sd[...] + bd[...]
        else:
            idn = cur[hal:hal + tm + 2 * rout, :].astype(jnp.float32)
        y = jnp.maximum(y + idn, 0.0)
        cur = jnp.where(ok, y, 0.0).astype(jnp.bfloat16)
    out[...] = cur


def _chain(x, geom, blocks, down=None):
    M, Cin = x.shape
    C4 = blocks[0][6].shape[1]
    tm = _TM
    nh = M // 256
    full = lambda a: pl.BlockSpec(a.shape, lambda i: (0, 0))
    in_specs = [
        pl.BlockSpec((256, Cin), lambda i: (jnp.maximum(i * 2 - 1, 0), 0)),
        pl.BlockSpec((tm, Cin), lambda i: (i, 0)),
        pl.BlockSpec((256, Cin), lambda i: (jnp.minimum(i * 2 + 2, nh - 1),
                                            0)),
    ]
    args = [x, x, x]
    for blk in blocks:
        args += list(blk)
        in_specs += [full(a) for a in blk]
    if down is not None:
        args += list(down)
        in_specs += [full(a) for a in down]
    return pl.pallas_call(
        functools.partial(_chain_body, geom=geom, nblocks=len(blocks),
                          has_down=down is not None),
        out_shape=jax.ShapeDtypeStruct((M, C4), jnp.bfloat16),
        grid=(M // tm,),
        in_specs=in_specs,
        out_specs=pl.BlockSpec((tm, C4), lambda i: (i, 0)),
        compiler_params=pltpu.CompilerParams(
            dimension_semantics=("parallel",),
            vmem_limit_bytes=_VMEM),
    )(*args)


def _pool_body(*refs):
    acc = refs[0][...]
    for r in refs[1:-1]:
        acc = jnp.maximum(acc, r[...])
    refs[-1][...] = acc


def _stem_body(pe_pv, pe_cu, pe_nx, po_pv, po_cu, po_nx, we_ref, wo_ref,
               s_ref, b_ref, o_ref, *, tm, W):
    pe = jnp.concatenate([pe_pv[7:, :], pe_cu[...], pe_nx[...]], axis=0)
    po = jnp.concatenate([po_pv[7:, :], po_cu[...], po_nx[...]], axis=0)
    he = jnp.concatenate([pe[e * W:e * W + tm + 2, :] for e in range(4)],
                         axis=1)
    ho = jnp.concatenate([po[e * W:e * W + tm + 2, :] for e in range(3)],
                         axis=1)
    y = (jnp.dot(he, we_ref[...], preferred_element_type=jnp.float32)
         + jnp.dot(ho, wo_ref[...], preferred_element_type=jnp.float32))
    y = jnp.maximum(y * s_ref[...] + b_ref[...], 0.0)
    wcol = jax.lax.rem(_rows_iota(tm, pl.program_id(0) * tm), W)
    left = jnp.where(wcol >= 1, y[0:tm, :], -jnp.inf)
    right = jnp.where(wcol <= W - 2, y[2:tm + 2, :], -jnp.inf)
    o_ref[...] = jnp.maximum(jnp.maximum(y[1:tm + 1, :], left),
                             right).astype(o_ref.dtype)


def _gap_proj_body(x_ref, w_ref, s_ref, b_ref, o_ref, *, hw):
    f = jnp.sum(x_ref[...].astype(jnp.float32), axis=1) * (1.0 / hw)
    y = jnp.dot(f.astype(jnp.bfloat16), w_ref[...],
                preferred_element_type=jnp.float32)
    o_ref[...] = y * s_ref[...] + b_ref[...]


def _geom(H, W, B):
    Hp, Wp = H + 2, W + 2
    img = Hp * Wp
    img_p = _cdiv(img, 16) * 16
    return {"H": H, "W": W, "Wp": Wp, "img": img, "img_p": img_p,
            "hal": Wp + 1, "M": B * img_p}


def _to_layout(x, geom):
    B, H, W, C = x.shape
    xp = jnp.pad(x, ((0, 0), (1, 1), (1, 1), (0, 0)))
    xp = xp.reshape(B, geom["img"], C)
    xp = jnp.pad(xp, ((0, 0), (0, geom["img_p"] - geom["img"]), (0, 0)))
    return xp.reshape(B * geom["img_p"], C)


def _from_layout(x, geom, B):
    C = x.shape[1]
    return (x.reshape(B, geom["img_p"], C)[:, :geom["img"], :]
            .reshape(B, geom["H"] + 2, geom["Wp"], C))


def _im2col_s2(xpad, Ho, Wo, k=3):
    cols = [xpad[:, dy:dy + 2 * Ho - 1:2, dx:dx + 2 * Wo - 1:2, :]
            for dy in range(k) for dx in range(k)]
    B = xpad.shape[0]
    return jnp.concatenate(cols, axis=-1).reshape(B * Ho * Wo, -1)


def _block_s2(x, gin, gout, B, p):
    (w1, s1, b1, w2, s2, b2, w3, s3, b3, wd, sd, bd) = p
    Ho, Wo = gout["H"], gout["W"]
    t1 = _mm(x, w1, s1, b1, relu=True, geom=gin)
    t1p = _from_layout(t1, gin, B)
    rows = _im2col_s2(t1p, Ho, Wo)
    t2 = _mm(rows, w2, s2, b2, relu=True)
    xc = _from_layout(x, gin, B)[:, 1:2 * Ho:2, 1:2 * Wo:2, :]
    idn = _mm(xc.reshape(B * Ho * Wo, -1), wd, sd, bd, relu=False)
    y = _mm(t2, w3, s3, b3, relu=True, res=idn)
    return _to_layout(y.reshape(B, Ho, Wo, -1), gout)


def kernel(images, conv1, bn1_s, bn1_b, L1B0_conv1, L1B0_conv2, L1B0_conv3, L1B0_s1, L1B0_b1, L1B0_s2, L1B0_b2, L1B0_s3, L1B0_b3, L1B0_down, L1B0_sd, L1B0_bd, L1B1_conv1, L1B1_conv2, L1B1_conv3, L1B1_s1, L1B1_b1, L1B1_s2, L1B1_b2, L1B1_s3, L1B1_b3, L1B2_conv1, L1B2_conv2, L1B2_conv3, L1B2_s1, L1B2_b1, L1B2_s2, L1B2_b2, L1B2_s3, L1B2_b3, L2B0_conv1, L2B0_conv2, L2B0_conv3, L2B0_s1, L2B0_b1, L2B0_s2, L2B0_b2, L2B0_s3, L2B0_b3, L2B0_down, L2B0_sd, L2B0_bd, L2B1_conv1, L2B1_conv2, L2B1_conv3, L2B1_s1, L2B1_b1, L2B1_s2, L2B1_b2, L2B1_s3, L2B1_b3, L2B2_conv1, L2B2_conv2, L2B2_conv3, L2B2_s1, L2B2_b1, L2B2_s2, L2B2_b2, L2B2_s3, L2B2_b3, L2B3_conv1, L2B3_conv2, L2B3_conv3, L2B3_s1, L2B3_b1, L2B3_s2, L2B3_b2, L2B3_s3, L2B3_b3, L3B0_conv1, L3B0_conv2, L3B0_conv3, L3B0_s1, L3B0_b1, L3B0_s2, L3B0_b2, L3B0_s3, L3B0_b3, L3B0_down, L3B0_sd, L3B0_bd, L3B1_conv1, L3B1_conv2, L3B1_conv3, L3B1_s1, L3B1_b1, L3B1_s2, L3B1_b2, L3B1_s3, L3B1_b3, L3B2_conv1, L3B2_conv2, L3B2_conv3, L3B2_s1, L3B2_b1, L3B2_s2, L3B2_b2, L3B2_s3, L3B2_b3, L3B3_conv1, L3B3_conv2, L3B3_conv3, L3B3_s1, L3B3_b1, L3B3_s2, L3B3_b2, L3B3_s3, L3B3_b3, L3B4_conv1, L3B4_conv2, L3B4_conv3, L3B4_s1, L3B4_b1, L3B4_s2, L3B4_b2, L3B4_s3, L3B4_b3, L3B5_conv1, L3B5_conv2, L3B5_conv3, L3B5_s1, L3B5_b1, L3B5_s2, L3B5_b2, L3B5_s3, L3B5_b3, L4B0_conv1, L4B0_conv2, L4B0_conv3, L4B0_s1, L4B0_b1, L4B0_s2, L4B0_b2, L4B0_s3, L4B0_b3, L4B0_down, L4B0_sd, L4B0_bd, L4B1_conv1, L4B1_conv2, L4B1_conv3, L4B1_s1, L4B1_b1, L4B1_s2, L4B1_b2, L4B1_s3, L4B1_b3, L4B2_conv1, L4B2_conv2, L4B2_conv3, L4B2_s1, L4B2_b1, L4B2_s2, L4B2_b2, L4B2_s3, L4B2_b3, proj_w, proj_s, proj_b):
    B = images.shape[0]
    g1, g2, g3, g4 = (_geom(56, 56, B), _geom(28, 28, B), _geom(14, 14, B),
                      _geom(7, 7, B))

    x = jnp.transpose(images, (0, 2, 3, 1)).astype(jnp.bfloat16)
    xp = jnp.pad(x, ((0, 0), (3, 3), (3, 3), (0, 0)))
    zpad = jnp.zeros((B, 115, 112, 3), jnp.bfloat16)

    def dx_rows(par):
        xs = xp[:, par:230:2, :, :]
        cols = [xs[:, :, dx:dx + 223:2, :] for dx in range(7)] + [zpad]
        return jnp.concatenate(cols, axis=-1).reshape(B * 115 * 112, 24)

    pe, po = dx_rows(0), dx_rows(1)
    idx_e = [(2 * e * 7 + dx) * 3 + c for e in range(4) for dx in range(7)
             for c in range(3)]
    idx_o = [((2 * e + 1) * 7 + dx) * 3 + c for e in range(3)
             for dx in range(7) for c in range(3)]
    pad3 = [147, 148, 149]
    w_e = conv1[jnp.array([k for e in range(4)
                           for k in idx_e[e * 21:e * 21 + 21] + pad3]), :]
    w_o = conv1[jnp.array([k for e in range(3)
                           for k in idx_o[e * 21:e * 21 + 21] + pad3]), :]

    M0 = B * 115 * 112
    ni0 = M0 // _TM
    hspec = pl.BlockSpec((8, 24), lambda i: (jnp.maximum(i * 64 - 1, 0), 0))
    cspec = pl.BlockSpec((_TM, 24), lambda i: (i, 0))
    nspec = pl.BlockSpec((_TM, 24), lambda i: (jnp.minimum(i + 1,
                                                           ni0 - 1), 0))
    wmax = pl.pallas_call(
        functools.partial(_stem_body, tm=_TM, W=112),
        out_shape=jax.ShapeDtypeStruct((M0, 128), jnp.bfloat16),
        grid=(ni0,),
        in_specs=[
            hspec, cspec, nspec, hspec, cspec, nspec,
            pl.BlockSpec((96, 128), lambda i: (0, 0)),
            pl.BlockSpec((72, 128), lambda i: (0, 0)),
            pl.BlockSpec((1, 128), lambda i: (0, 0)),
            pl.BlockSpec((1, 128), lambda i: (0, 0)),
        ],
        out_specs=pl.BlockSpec((_TM, 128), lambda i: (i, 0)),
        compiler_params=pltpu.CompilerParams(
            dimension_semantics=("parallel",), vmem_limit_bytes=_VMEM),
    )(pe, pe, pe, po, po, po, w_e, w_o, bn1_s, bn1_b)

    yp = jnp.pad(wmax.reshape(B, 115, 112, 128), ((0, 0), (1, 0), (0, 0),
                                                  (0, 0)),
                 constant_values=-jnp.inf)
    taps = [yp[:, dy:dy + 111:2, 0:112:2, :].reshape(B * 56 * 56, 128)
            for dy in range(3)]
    M1 = B * 56 * 56
    pooled = pl.pallas_call(
        _pool_body,
        out_shape=jax.ShapeDtypeStruct((M1, 128), jnp.bfloat16),
        grid=(M1 // _TM,),
        in_specs=[pl.BlockSpec((_TM, 128), lambda i: (i, 0))] * 3,
        out_specs=pl.BlockSpec((_TM, 128), lambda i: (i, 0)),
        compiler_params=pltpu.CompilerParams(
            dimension_semantics=("parallel",), vmem_limit_bytes=_VMEM),
    )(*taps)
    x = _to_layout(pooled.reshape(B, 56, 56, 128), g1)

    x = _chain(x, g1,
               [(L1B0_conv1, L1B0_s1, L1B0_b1, L1B0_conv2, L1B0_s2, L1B0_b2,
                 L1B0_conv3, L1B0_s3, L1B0_b3)],
               down=(L1B0_down, L1B0_sd, L1B0_bd))
    x = _chain(x, g1,
               [(L1B1_conv1, L1B1_s1, L1B1_b1, L1B1_conv2, L1B1_s2, L1B1_b2,
                 L1B1_conv3, L1B1_s3, L1B1_b3)])
    x = _chain(x, g1,
               [(L1B2_conv1, L1B2_s1, L1B2_b1, L1B2_conv2, L1B2_s2, L1B2_b2,
                 L1B2_conv3, L1B2_s3, L1B2_b3)])

    x = _block_s2(x, g1, g2, B, (L2B0_conv1, L2B0_s1, L2B0_b1, L2B0_conv2,
                                 L2B0_s2, L2B0_b2, L2B0_conv3, L2B0_s3,
                                 L2B0_b3, L2B0_down, L2B0_sd, L2B0_bd))
    x = _chain(x, g2,
               [(L2B1_conv1, L2B1_s1, L2B1_b1, L2B1_conv2, L2B1_s2, L2B1_b2,
                 L2B1_conv3, L2B1_s3, L2B1_b3),
                (L2B2_conv1, L2B2_s1, L2B2_b1, L2B2_conv2, L2B2_s2, L2B2_b2,
                 L2B2_conv3, L2B2_s3, L2B2_b3),
                (L2B3_conv1, L2B3_s1, L2B3_b1, L2B3_conv2, L2B3_s2, L2B3_b2,
                 L2B3_conv3, L2B3_s3, L2B3_b3)])

    x = _block_s2(x, g2, g3, B, (L3B0_conv1, L3B0_s1, L3B0_b1, L3B0_conv2,
                                 L3B0_s2, L3B0_b2, L3B0_conv3, L3B0_s3,
                                 L3B0_b3, L3B0_down, L3B0_sd, L3B0_bd))
    x = _chain(x, g3,
               [(L3B1_conv1, L3B1_s1, L3B1_b1, L3B1_conv2, L3B1_s2, L3B1_b2,
                 L3B1_conv3, L3B1_s3, L3B1_b3),
                (L3B2_conv1, L3B2_s1, L3B2_b1, L3B2_conv2, L3B2_s2, L3B2_b2,
                 L3B2_conv3, L3B2_s3, L3B2_b3),
                (L3B3_conv1, L3B3_s1, L3B3_b1, L3B3_conv2, L3B3_s2, L3B3_b2,
                 L3B3_conv3, L3B3_s3, L3B3_b3),
                (L3B4_conv1, L3B4_s1, L3B4_b1, L3B4_conv2, L3B4_s2, L3B4_b2,
                 L3B4_conv3, L3B4_s3, L3B4_b3),
                (L3B5_conv1, L3B5_s1, L3B5_b1, L3B5_conv2, L3B5_s2, L3B5_b2,
                 L3B5_conv3, L3B5_s3, L3B5_b3)])

    x = _block_s2(x, g3, g4, B, (L4B0_conv1, L4B0_s1, L4B0_b1, L4B0_conv2,
                                 L4B0_s2, L4B0_b2, L4B0_conv3, L4B0_s3,
                                 L4B0_b3, L4B0_down, L4B0_sd, L4B0_bd))
    x = _chain(x, g4,
               [(L4B1_conv1, L4B1_s1, L4B1_b1, L4B1_conv2, L4B1_s2, L4B1_b2,
                 L4B1_conv3, L4B1_s3, L4B1_b3),
                (L4B2_conv1, L4B2_s1, L4B2_b1, L4B2_conv2, L4B2_s2, L4B2_b2,
                 L4B2_conv3, L4B2_s3, L4B2_b3)])

    x3 = x.reshape(B, g4["img_p"], 2048)
    out = pl.pallas_call(
        functools.partial(_gap_proj_body, hw=49.0),
        out_shape=jax.ShapeDtypeStruct((B, 512), jnp.float32),
        compiler_params=pltpu.CompilerParams(vmem_limit_bytes=_VMEM),
    )(x3, proj_w, proj_s, proj_b)
    return out.reshape(B, 1, 512)


# speedup vs baseline: 1.8863x; 1.1660x over previous
import functools

import jax
import jax.numpy as jnp
from jax.experimental import pallas as pl
from jax.experimental.pallas import tpu as pltpu

_TM = 512
_VMEM = 100 * 1024 * 1024


def _cdiv(a, b):
    return (a + b - 1) // b


def _interior_mask(g, geom):
    r = jax.lax.rem(g, geom["img_p"])
    w = jax.lax.rem(r, geom["Wp"])
    ok = ((r >= geom["Wp"]) & (r < (geom["H"] + 1) * geom["Wp"])
          & (w >= 1) & (w <= geom["W"]))
    if "M" in geom:
        ok &= g < geom["M"]
    return ok


def _rows_iota(n, base):
    return jax.lax.broadcasted_iota(jnp.int32, (n, 1), 0) + base


def _mm_body(*refs, relu, has_res, geom, tm):
    if has_res:
        x_ref, w_ref, s_ref, b_ref, r_ref, o_ref = refs
    else:
        x_ref, w_ref, s_ref, b_ref, o_ref = refs
    y = jnp.dot(x_ref[...], w_ref[...], preferred_element_type=jnp.float32)
    y = y * s_ref[...] + b_ref[...]
    if has_res:
        y = y + r_ref[...].astype(jnp.float32)
    if relu:
        y = jnp.maximum(y, 0.0)
    if geom is not None:
        g = _rows_iota(y.shape[0], pl.program_id(0) * tm)
        y = jnp.where(_interior_mask(g, geom), y, 0.0)
    o_ref[...] = y.astype(o_ref.dtype)


def _mm(x, w, s, b, relu, res=None, out_dtype=jnp.bfloat16, geom=None):
    M, K = x.shape
    N = w.shape[1]
    tm = min(_TM, M)
    tn = min(N, 512)
    grid = (_cdiv(M, tm), N // tn)
    in_specs = [
        pl.BlockSpec((tm, K), lambda i, j: (i, 0)),
        pl.BlockSpec((K, tn), lambda i, j: (0, j)),
        pl.BlockSpec((1, tn), lambda i, j: (0, j)),
        pl.BlockSpec((1, tn), lambda i, j: (0, j)),
    ]
    args = [x.astype(jnp.bfloat16), w, s, b]
    if res is not None:
        in_specs.append(pl.BlockSpec((tm, tn), lambda i, j: (i, j)))
        args.append(res.astype(jnp.bfloat16))
    return pl.pallas_call(
        functools.partial(_mm_body, relu=relu, has_res=res is not None,
                          geom=geom, tm=tm),
        out_shape=jax.ShapeDtypeStruct((M, N), out_dtype),
        grid=grid,
        in_specs=in_specs,
        out_specs=pl.BlockSpec((tm, tn), lambda i, j: (i, j)),
        compiler_params=pltpu.CompilerParams(
            dimension_semantics=("parallel", "parallel"),
            vmem_limit_bytes=_VMEM),
    )(*args)


def _chain_body(*refs, geom, nblocks, has_down):
    pv, cu, nx = refs[:3]
    out = refs[-1]
    tm = cu.shape[0]
    hal = geom["hal"]
    base0 = pl.program_id(0) * tm
    win = jnp.concatenate([pv[...], cu[...], nx[...]], axis=0)
    rin0 = nblocks * hal
    cur = win[256 - rin0:256 + tm + rin0, :]
    x0 = cur
    idx = 3
    for j in range(nblocks):
        rin = (nblocks - j) * hal
        rout = rin - hal
        w1, s1, b1, w2, s2, b2, w3, s3, b3 = refs[idx:idx + 9]
        idx += 9
        C = w1.shape[1]
        t1 = jnp.dot(cur, w1[...], preferred_element_type=jnp.float32)
        t1 = jnp.maximum(t1 * s1[...] + b1[...], 0.0)
        t1 = jnp.where(
            _interior_mask(_rows_iota(tm + 2 * rin, base0 - rin), geom),
            t1, 0.0).astype(jnp.bfloat16)
        acc = None
        for dy in range(3):
            for dx in range(3):
                d = dy * geom["Wp"] + dx - hal
                t = dy * 3 + dx
                p = jnp.dot(t1[hal + d:hal + d + tm + 2 * rout, :],
                            w2[t * C:(t + 1) * C, :],
                            preferred_element_type=jnp.float32)
                acc = p if acc is None else acc + p
        ok = _interior_mask(_rows_iota(tm + 2 * rout, base0 - rout), geom)
        t2 = jnp.where(ok, jnp.maximum(acc * s2[...] + b2[...], 0.0),
                       0.0).astype(jnp.bfloat16)
        y = jnp.dot(t2, w3[...], preferred_element_type=jnp.float32)
        y = y * s3[...] + b3[...]
        if j == 0 and has_down:
            wd, sd, bd = refs[-4], refs[-3], refs[-2]
            idn = jnp.dot(x0[hal:hal + tm + 2 * rout, :], wd[...],
                          preferred_element_type=jnp.float32)
            idn = idn * sd[...] + bd[...]
        else:
            idn = cur[hal:hal + tm + 2 * rout, :].astype(jnp.float32)
        y = jnp.maximum(y + idn, 0.0)
        cur = jnp.where(ok, y, 0.0).astype(jnp.bfloat16)
    out[...] = cur


def _chain(x, geom, blocks, down=None):
    M, Cin = x.shape
    C4 = blocks[0][6].shape[1]
    tm = _TM
    nh = M // 256
    full = lambda a: pl.BlockSpec(a.shape, lambda i: (0, 0))
    in_specs = [
        pl.BlockSpec((256, Cin), lambda i: (jnp.maximum(i * 2 - 1, 0), 0)),
        pl.BlockSpec((tm, Cin), lambda i: (i, 0)),
        pl.BlockSpec((256, Cin), lambda i: (jnp.minimum(i * 2 + 2, nh - 1),
                                            0)),
    ]
    args = [x, x, x]
    for blk in blocks:
        args += list(blk)
        in_specs += [full(a) for a in blk]
    if down is not None:
        args += list(down)
        in_specs += [full(a) for a in down]
    return pl.pallas_call(
        functools.partial(_chain_body, geom=geom, nblocks=len(blocks),
                          has_down=down is not None),
        out_shape=jax.ShapeDtypeStruct((M, C4), jnp.bfloat16),
        grid=(M // tm,),
        in_specs=in_specs,
        out_specs=pl.BlockSpec((tm, C4), lambda i: (i, 0)),
        compiler_params=pltpu.CompilerParams(
            dimension_semantics=("parallel",),
            vmem_limit_bytes=_VMEM),
    )(*args)


def _pool_body(*refs):
    acc = refs[0][...]
    for r in refs[1:-1]:
        acc = jnp.maximum(acc, r[...])
    refs[-1][...] = acc


def _stem_body(pe_pv, pe_cu, pe_nx, po_pv, po_cu, po_nx, we_ref, wo_ref,
               s_ref, b_ref, o_ref, *, tm, W):
    pe = jnp.concatenate([pe_pv[7:, :], pe_cu[...], pe_nx[...]], axis=0)
    po = jnp.concatenate([po_pv[7:, :], po_cu[...], po_nx[...]], axis=0)
    he = jnp.concatenate([pe[e * W:e * W + tm + 2, :] for e in range(4)],
                         axis=1)
    ho = jnp.concatenate([po[e * W:e * W + tm + 2, :] for e in range(3)],
                         axis=1)
    y = (jnp.dot(he, we_ref[...], preferred_element_type=jnp.float32)
         + jnp.dot(ho, wo_ref[...], preferred_element_type=jnp.float32))
    y = jnp.maximum(y * s_ref[...] + b_ref[...], 0.0)
    wcol = jax.lax.rem(_rows_iota(tm, pl.program_id(0) * tm), W)
    left = jnp.where(wcol >= 1, y[0:tm, :], -jnp.inf)
    right = jnp.where(wcol <= W - 2, y[2:tm + 2, :], -jnp.inf)
    o_ref[...] = jnp.maximum(jnp.maximum(y[1:tm + 1, :], left),
                             right).astype(o_ref.dtype)


def _gap_proj_body(x_ref, w_ref, s_ref, b_ref, o_ref, *, hw):
    f = jnp.sum(x_ref[...].astype(jnp.float32), axis=1) * (1.0 / hw)
    y = jnp.dot(f.astype(jnp.bfloat16), w_ref[...],
                preferred_element_type=jnp.float32)
    o_ref[...] = y * s_ref[...] + b_ref[...]


def _geom(H, W, B):
    Hp, Wp = H + 2, W + 2
    img = Hp * Wp
    img_p = _cdiv(img, 16) * 16
    return {"H": H, "W": W, "Wp": Wp, "img": img, "img_p": img_p,
            "hal": Wp + 1, "M": B * img_p}


def _to_layout(x, geom):
    B, H, W, C = x.shape
    xp = jnp.pad(x, ((0, 0), (1, 1), (1, 1), (0, 0)))
    xp = xp.reshape(B, geom["img"], C)
    xp = jnp.pad(xp, ((0, 0), (0, geom["img_p"] - geom["img"]), (0, 0)))
    return xp.reshape(B * geom["img_p"], C)


def _from_layout(x, geom, B):
    C = x.shape[1]
    return (x.reshape(B, geom["img_p"], C)[:, :geom["img"], :]
            .reshape(B, geom["H"] + 2, geom["Wp"], C))


def _im2col_s2(xpad, Ho, Wo, k=3):
    cols = [xpad[:, dy:dy + 2 * Ho - 1:2, dx:dx + 2 * Wo - 1:2, :]
            for dy in range(k) for dx in range(k)]
    B = xpad.shape[0]
    return jnp.concatenate(cols, axis=-1).reshape(B * Ho * Wo, -1)


def _block_s2(x, gin, gout, B, p):
    (w1, s1, b1, w2, s2, b2, w3, s3, b3, wd, sd, bd) = p
    Ho, Wo = gout["H"], gout["W"]
    t1 = _mm(x, w1, s1, b1, relu=True, geom=gin)
    t1p = _from_layout(t1, gin, B)
    rows = _im2col_s2(t1p, Ho, Wo)
    t2 = _mm(rows, w2, s2, b2, relu=True)
    xc = _from_layout(x, gin, B)[:, 1:2 * Ho:2, 1:2 * Wo:2, :]
    idn = _mm(xc.reshape(B * Ho * Wo, -1), wd, sd, bd, relu=False)
    y = _mm(t2, w3, s3, b3, relu=True, res=idn)
    return _to_layout(y.reshape(B, Ho, Wo, -1), gout)


def kernel(images, conv1, bn1_s, bn1_b, L1B0_conv1, L1B0_conv2, L1B0_conv3, L1B0_s1, L1B0_b1, L1B0_s2, L1B0_b2, L1B0_s3, L1B0_b3, L1B0_down, L1B0_sd, L1B0_bd, L1B1_conv1, L1B1_conv2, L1B1_conv3, L1B1_s1, L1B1_b1, L1B1_s2, L1B1_b2, L1B1_s3, L1B1_b3, L1B2_conv1, L1B2_conv2, L1B2_conv3, L1B2_s1, L1B2_b1, L1B2_s2, L1B2_b2, L1B2_s3, L1B2_b3, L2B0_conv1, L2B0_conv2, L2B0_conv3, L2B0_s1, L2B0_b1, L2B0_s2, L2B0_b2, L2B0_s3, L2B0_b3, L2B0_down, L2B0_sd, L2B0_bd, L2B1_conv1, L2B1_conv2, L2B1_conv3, L2B1_s1, L2B1_b1, L2B1_s2, L2B1_b2, L2B1_s3, L2B1_b3, L2B2_conv1, L2B2_conv2, L2B2_conv3, L2B2_s1, L2B2_b1, L2B2_s2, L2B2_b2, L2B2_s3, L2B2_b3, L2B3_conv1, L2B3_conv2, L2B3_conv3, L2B3_s1, L2B3_b1, L2B3_s2, L2B3_b2, L2B3_s3, L2B3_b3, L3B0_conv1, L3B0_conv2, L3B0_conv3, L3B0_s1, L3B0_b1, L3B0_s2, L3B0_b2, L3B0_s3, L3B0_b3, L3B0_down, L3B0_sd, L3B0_bd, L3B1_conv1, L3B1_conv2, L3B1_conv3, L3B1_s1, L3B1_b1, L3B1_s2, L3B1_b2, L3B1_s3, L3B1_b3, L3B2_conv1, L3B2_conv2, L3B2_conv3, L3B2_s1, L3B2_b1, L3B2_s2, L3B2_b2, L3B2_s3, L3B2_b3, L3B3_conv1, L3B3_conv2, L3B3_conv3, L3B3_s1, L3B3_b1, L3B3_s2, L3B3_b2, L3B3_s3, L3B3_b3, L3B4_conv1, L3B4_conv2, L3B4_conv3, L3B4_s1, L3B4_b1, L3B4_s2, L3B4_b2, L3B4_s3, L3B4_b3, L3B5_conv1, L3B5_conv2, L3B5_conv3, L3B5_s1, L3B5_b1, L3B5_s2, L3B5_b2, L3B5_s3, L3B5_b3, L4B0_conv1, L4B0_conv2, L4B0_conv3, L4B0_s1, L4B0_b1, L4B0_s2, L4B0_b2, L4B0_s3, L4B0_b3, L4B0_down, L4B0_sd, L4B0_bd, L4B1_conv1, L4B1_conv2, L4B1_conv3, L4B1_s1, L4B1_b1, L4B1_s2, L4B1_b2, L4B1_s3, L4B1_b3, L4B2_conv1, L4B2_conv2, L4B2_conv3, L4B2_s1, L4B2_b1, L4B2_s2, L4B2_b2, L4B2_s3, L4B2_b3, proj_w, proj_s, proj_b):
    B = images.shape[0]
    g1, g2, g3, g4 = (_geom(56, 56, B), _geom(28, 28, B), _geom(14, 14, B),
                      _geom(7, 7, B))

    x = jnp.transpose(images, (0, 2, 3, 1)).astype(jnp.bfloat16)
    xp = jnp.pad(x, ((0, 0), (3, 3), (3, 3), (0, 0)))
    zpad = jnp.zeros((B, 115, 112, 3), jnp.bfloat16)

    def dx_rows(par):
        xs = xp[:, par:230:2, :, :]
        cols = [xs[:, :, dx:dx + 223:2, :] for dx in range(7)] + [zpad]
        return jnp.concatenate(cols, axis=-1).reshape(B * 115 * 112, 24)

    pe, po = dx_rows(0), dx_rows(1)
    idx_e = [(2 * e * 7 + dx) * 3 + c for e in range(4) for dx in range(7)
             for c in range(3)]
    idx_o = [((2 * e + 1) * 7 + dx) * 3 + c for e in range(3)
             for dx in range(7) for c in range(3)]
    pad3 = [147, 148, 149]
    w_e = conv1[jnp.array([k for e in range(4)
                           for k in idx_e[e * 21:e * 21 + 21] + pad3]), :]
    w_o = conv1[jnp.array([k for e in range(3)
                           for k in idx_o[e * 21:e * 21 + 21] + pad3]), :]

    M0 = B * 115 * 112
    ni0 = M0 // _TM
    hspec = pl.BlockSpec((8, 24), lambda i: (jnp.maximum(i * 64 - 1, 0), 0))
    cspec = pl.BlockSpec((_TM, 24), lambda i: (i, 0))
    nspec = pl.BlockSpec((_TM, 24), lambda i: (jnp.minimum(i + 1,
                                                           ni0 - 1), 0))
    wmax = pl.pallas_call(
        functools.partial(_stem_body, tm=_TM, W=112),
        out_shape=jax.ShapeDtypeStruct((M0, 128), jnp.bfloat16),
        grid=(ni0,),
        in_specs=[
            hspec, cspec, nspec, hspec, cspec, nspec,
            pl.BlockSpec((96, 128), lambda i: (0, 0)),
            pl.BlockSpec((72, 128), lambda i: (0, 0)),
            pl.BlockSpec((1, 128), lambda i: (0, 0)),
            pl.BlockSpec((1, 128), lambda i: (0, 0)),
        ],
        out_specs=pl.BlockSpec((_TM, 128), lambda i: (i, 0)),
        compiler_params=pltpu.CompilerParams(
            dimension_semantics=("parallel",), vmem_limit_bytes=_VMEM),
    )(pe, pe, pe, po, po, po, w_e, w_o, bn1_s, bn1_b)

    wm = wmax.reshape(B, 115, 112, 128)
    ninf = jnp.full((B, 1, 112, 128), -jnp.inf, jnp.bfloat16)
    taps = [jnp.concatenate([ninf, wm[:, 1:110:2]], axis=1),
            wm[:, 0:111:2], wm[:, 1:112:2]]
    taps = [t[:, :, 0:112:2, :].reshape(B * 56 * 56, 128) for t in taps]
    M1 = B * 56 * 56
    pooled = pl.pallas_call(
        _pool_body,
        out_shape=jax.ShapeDtypeStruct((M1, 128), jnp.bfloat16),
        grid=(M1 // _TM,),
        in_specs=[pl.BlockSpec((_TM, 128), lambda i: (i, 0))] * 3,
        out_specs=pl.BlockSpec((_TM, 128), lambda i: (i, 0)),
        compiler_params=pltpu.CompilerParams(
            dimension_semantics=("parallel",), vmem_limit_bytes=_VMEM),
    )(*taps)
    x = _to_layout(pooled.reshape(B, 56, 56, 128), g1)

    x = _chain(x, g1,
               [(L1B0_conv1, L1B0_s1, L1B0_b1, L1B0_conv2, L1B0_s2, L1B0_b2,
                 L1B0_conv3, L1B0_s3, L1B0_b3)],
               down=(L1B0_down, L1B0_sd, L1B0_bd))
    x = _chain(x, g1,
               [(L1B1_conv1, L1B1_s1, L1B1_b1, L1B1_conv2, L1B1_s2, L1B1_b2,
                 L1B1_conv3, L1B1_s3, L1B1_b3)])
    x = _chain(x, g1,
               [(L1B2_conv1, L1B2_s1, L1B2_b1, L1B2_conv2, L1B2_s2, L1B2_b2,
                 L1B2_conv3, L1B2_s3, L1B2_b3)])

    x = _block_s2(x, g1, g2, B, (L2B0_conv1, L2B0_s1, L2B0_b1, L2B0_conv2,
                                 L2B0_s2, L2B0_b2, L2B0_conv3, L2B0_s3,
                                 L2B0_b3, L2B0_down, L2B0_sd, L2B0_bd))
    x = _chain(x, g2,
               [(L2B1_conv1, L2B1_s1, L2B1_b1, L2B1_conv2, L2B1_s2, L2B1_b2,
                 L2B1_conv3, L2B1_s3, L2B1_b3),
                (L2B2_conv1, L2B2_s1, L2B2_b1, L2B2_conv2, L2B2_s2, L2B2_b2,
                 L2B2_conv3, L2B2_s3, L2B2_b3),
                (L2B3_conv1, L2B3_s1, L2B3_b1, L2B3_conv2, L2B3_s2, L2B3_b2,
                 L2B3_conv3, L2B3_s3, L2B3_b3)])

    x = _block_s2(x, g2, g3, B, (L3B0_conv1, L3B0_s1, L3B0_b1, L3B0_conv2,
                                 L3B0_s2, L3B0_b2, L3B0_conv3, L3B0_s3,
                                 L3B0_b3, L3B0_down, L3B0_sd, L3B0_bd))
    x = _chain(x, g3,
               [(L3B1_conv1, L3B1_s1, L3B1_b1, L3B1_conv2, L3B1_s2, L3B1_b2,
                 L3B1_conv3, L3B1_s3, L3B1_b3),
                (L3B2_conv1, L3B2_s1, L3B2_b1, L3B2_conv2, L3B2_s2, L3B2_b2,
                 L3B2_conv3, L3B2_s3, L3B2_b3),
                (L3B3_conv1, L3B3_s1, L3B3_b1, L3B3_conv2, L3B3_s2, L3B3_b2,
                 L3B3_conv3, L3B3_s3, L3B3_b3),
                (L3B4_conv1, L3B4_s1, L3B4_b1, L3B4_conv2, L3B4_s2, L3B4_b2,
                 L3B4_conv3, L3B4_s3, L3B4_b3),
                (L3B5_conv1, L3B5_s1, L3B5_b1, L3B5_conv2, L3B5_s2, L3B5_b2,
                 L3B5_conv3, L3B5_s3, L3B5_b3)])

    x = _block_s2(x, g3, g4, B, (L4B0_conv1, L4B0_s1, L4B0_b1, L4B0_conv2,
                                 L4B0_s2, L4B0_b2, L4B0_conv3, L4B0_s3,
                                 L4B0_b3, L4B0_down, L4B0_sd, L4B0_bd))
    x = _chain(x, g4,
               [(L4B1_conv1, L4B1_s1, L4B1_b1, L4B1_conv2, L4B1_s2, L4B1_b2,
                 L4B1_conv3, L4B1_s3, L4B1_b3),
                (L4B2_conv1, L4B2_s1, L4B2_b1, L4B2_conv2, L4B2_s2, L4B2_b2,
                 L4B2_conv3, L4B2_s3, L4B2_b3)])

    x3 = x.reshape(B, g4["img_p"], 2048)
    out = pl.pallas_call(
        functools.partial(_gap_proj_body, hw=49.0),
        out_shape=jax.ShapeDtypeStruct((B, 512), jnp.float32),
        compiler_params=pltpu.CompilerParams(vmem_limit_bytes=_VMEM),
    )(x3, proj_w, proj_s, proj_b)
    return out.reshape(B, 1, 512)
